# Initial kernel scaffold; baseline (speedup 1.0000x reference)
import jax
import jax.numpy as jnp
from jax.experimental import pallas as pl


def kernel(a, b, coeffs):
    raise NotImplementedError("write your pallas kernel here")


# speedup vs baseline: 9.7432x; 9.7432x over previous
import jax
import jax.numpy as jnp
from jax import lax
from jax.experimental import pallas as pl
from jax.experimental.pallas import tpu as pltpu
from jax.experimental.pallas import tpu_sc as plsc

_GRID = 16
_STRIDE = 16
_BATCH = 16384
_NC, _NS, _L = 2, 16, 16
_NW = _NC * _NS
_BPW = _BATCH // _NW
_TAB = _GRID * _GRID


def _spline_body(a_hbm, b_hbm, base_hbm, sa_hbm, sb_hbm, out_hbm,
                 a_v, b_v, base_v, sa_v, sb_v, out_v):
    wid = lax.axis_index("s") * _NC + lax.axis_index("c")
    off = wid * _BPW
    pltpu.sync_copy(a_hbm.at[pl.ds(off, _BPW)], a_v)
    pltpu.sync_copy(b_hbm.at[pl.ds(off, _BPW)], b_v)
    pltpu.sync_copy(base_hbm, base_v)
    pltpu.sync_copy(sa_hbm, sa_v)
    pltpu.sync_copy(sb_hbm, sb_v)
    for j in range(_BPW // _L):
        av = a_v[pl.ds(j * _L, _L)]
        bv = b_v[pl.ds(j * _L, _L)]
        ia = jnp.minimum(lax.shift_right_logical(av, 4), _GRID - 1)
        ib = jnp.minimum(lax.shift_right_logical(bv, 4), _GRID - 1)
        idx = ia * _GRID + ib
        offa = (av & (_STRIDE - 1)).astype(jnp.float32)
        offb = (bv & (_STRIDE - 1)).astype(jnp.float32)
        c0 = plsc.load_gather(base_v, [idx])
        c1 = plsc.load_gather(sa_v, [idx])
        c2 = plsc.load_gather(sb_v, [idx])
        out_v[pl.ds(j * _L, _L)] = c0 + c1 * offa + c2 * offb
    pltpu.sync_copy(out_v, out_hbm.at[pl.ds(off, _BPW)])


def kernel(a, b, coeffs):
    cf = coeffs.reshape(_TAB, 3)
    run = pl.kernel(
        _spline_body,
        out_type=jax.ShapeDtypeStruct((_BATCH,), jnp.float32),
        mesh=plsc.VectorSubcoreMesh(core_axis_name="c", subcore_axis_name="s"),
        compiler_params=pltpu.CompilerParams(needs_layout_passes=False),
        scratch_types=[
            pltpu.VMEM((_BPW,), jnp.int32),
            pltpu.VMEM((_BPW,), jnp.int32),
            pltpu.VMEM((_TAB,), jnp.float32),
            pltpu.VMEM((_TAB,), jnp.float32),
            pltpu.VMEM((_TAB,), jnp.float32),
            pltpu.VMEM((_BPW,), jnp.float32),
        ],
    )
    out = run(a.astype(jnp.int32), b.astype(jnp.int32),
              cf[:, 0], cf[:, 1], cf[:, 2])
    return out.reshape(_BATCH, 1)


# speedup vs baseline: 10.6076x; 1.0887x over previous
import jax
import jax.numpy as jnp
from jax import lax
from jax.experimental import pallas as pl
from jax.experimental.pallas import tpu as pltpu
from jax.experimental.pallas import tpu_sc as plsc

_GRID = 16
_STRIDE = 16
_BATCH = 16384
_NC, _NS, _L = 2, 16, 16
_NW = _NC * _NS
_BPW = _BATCH // _NW
_TAB = _GRID * _GRID * 3


def _spline_body(a_hbm, b_hbm, tab_hbm, out_hbm, a_v, b_v, tab_v, out_v, sem):
    wid = lax.axis_index("s") * _NC + lax.axis_index("c")
    off = wid * _BPW
    copies = [
        pltpu.async_copy(a_hbm.at[pl.ds(off, _BPW)], a_v, sem),
        pltpu.async_copy(b_hbm.at[pl.ds(off, _BPW)], b_v, sem),
        pltpu.async_copy(tab_hbm, tab_v, sem),
    ]
    for c in copies:
        c.wait()
    for j in range(_BPW // _L):
        av = a_v[pl.ds(j * _L, _L)]
        bv = b_v[pl.ds(j * _L, _L)]
        ia = jnp.minimum(lax.shift_right_logical(av, 4), _GRID - 1)
        ib = jnp.minimum(lax.shift_right_logical(bv, 4), _GRID - 1)
        idx = (ia * _GRID + ib) * 3
        offa = (av & (_STRIDE - 1)).astype(jnp.float32)
        offb = (bv & (_STRIDE - 1)).astype(jnp.float32)
        c0 = plsc.load_gather(tab_v, [idx])
        c1 = plsc.load_gather(tab_v, [idx + 1])
        c2 = plsc.load_gather(tab_v, [idx + 2])
        out_v[pl.ds(j * _L, _L)] = c0 + c1 * offa + c2 * offb
    pltpu.sync_copy(out_v, out_hbm.at[pl.ds(off, _BPW)])


def kernel(a, b, coeffs):
    run = pl.kernel(
        _spline_body,
        out_type=jax.ShapeDtypeStruct((_BATCH,), jnp.float32),
        mesh=plsc.VectorSubcoreMesh(core_axis_name="c", subcore_axis_name="s"),
        compiler_params=pltpu.CompilerParams(needs_layout_passes=False),
        scratch_types=[
            pltpu.VMEM((_BPW,), jnp.int32),
            pltpu.VMEM((_BPW,), jnp.int32),
            pltpu.VMEM((_TAB,), jnp.float32),
            pltpu.VMEM((_BPW,), jnp.float32),
            pltpu.SemaphoreType.DMA,
        ],
    )
    out = run(a.astype(jnp.int32), b.astype(jnp.int32), coeffs.reshape(_TAB))
    return out.reshape(_BATCH, 1)
